# Initial kernel scaffold; baseline (speedup 1.0000x reference)
import jax
import jax.numpy as jnp
from jax.experimental import pallas as pl


def kernel(z):
    raise NotImplementedError("write your pallas kernel here")


# speedup vs baseline: 1.0725x; 1.0725x over previous
import functools

import jax
import jax.numpy as jnp
from jax import lax
from jax.experimental import pallas as pl
from jax.experimental.pallas import tpu as pltpu
from jax.experimental.pallas import tpu_sc as plsc

_N_WORDS = 8192
_WORD_DIM = 256
_NC = 2
_NS = 16
_NW = _NC * _NS
_ROWS_PER_W = _N_WORDS // _NW
_CHUNK = 128
_NCHUNKS = _ROWS_PER_W // _CHUNK


def _sc_gather(table, idx2d):
    mesh = plsc.VectorSubcoreMesh(core_axis_name="c", subcore_axis_name="s")

    @functools.partial(
        pl.kernel,
        mesh=mesh,
        out_type=jax.ShapeDtypeStruct((_N_WORDS, _WORD_DIM), jnp.float32),
        scratch_types=[
            pltpu.VMEM((_NCHUNKS, _CHUNK), jnp.int32),
            pltpu.VMEM((_ROWS_PER_W, _WORD_DIM), jnp.float32),
            pltpu.SemaphoreType.DMA,
        ],
    )
    def k(table_hbm, idx_hbm, out_hbm, idx_v, rows_v, sem):
        wid = lax.axis_index("s") * _NC + lax.axis_index("c")
        base = wid * _ROWS_PER_W
        pltpu.sync_copy(idx_hbm.at[pl.ds(wid * _NCHUNKS, _NCHUNKS)], idx_v)
        copies = []
        for j in range(_NCHUNKS):
            copies.append(
                pltpu.async_copy(
                    table_hbm.at[idx_v.at[j]],
                    rows_v.at[pl.ds(j * _CHUNK, _CHUNK)],
                    sem,
                )
            )
        for c in copies:
            c.wait()
        pltpu.sync_copy(rows_v, out_hbm.at[pl.ds(base, _ROWS_PER_W)])

    return k(table, idx2d)


def kernel(z):
    perm = jax.random.permutation(jax.random.key(1), z.shape[0])
    idx = perm[:_N_WORDS].astype(jnp.int32).reshape(_NW * _NCHUNKS, _CHUNK)
    return _sc_gather(z, idx)
